# baseline (speedup 1.0000x reference)
import dataclasses
import functools

import jax
import jax.numpy as jnp
from jax.experimental import pallas as pl
from jax.experimental.pallas import tpu as pltpu
from jax.experimental.pallas import tpu_sc as plsc

VOCAB_SIZE = 1000000
EMB_DIM = 64
CTX = 200
HID = 128
FLAT = CTX * EMB_DIM

V_TILE = 20480
N_TC_TILES = 46
SPLIT = N_TC_TILES * V_TILE
SC_ROWS = VOCAB_SIZE - SPLIT
SC_RB = 64
SC_STEPS = SC_ROWS // SC_RB


def _prep_body(idx_ref, embt_ref, oh_ref, w1_ref, b1_ref, h_ref, e2_ref, sem):
    def issue(t, carry):
        r = idx_ref[t]
        start = pl.multiple_of((r // 128) * 128, 128)
        pltpu.make_async_copy(
            embt_ref.at[:, pl.ds(start, 128)],
            e2_ref.at[pl.ds(t * EMB_DIM, EMB_DIM), :],
            sem,
        ).start()
        return carry

    jax.lax.fori_loop(0, CTX, issue, 0)

    def drain(t, carry):
        pltpu.make_async_copy(
            embt_ref.at[:, pl.ds(0, 128)],
            e2_ref.at[pl.ds(t * EMB_DIM, EMB_DIM), :],
            sem,
        ).wait()
        return carry

    jax.lax.fori_loop(0, CTX, drain, 0)

    esel = jnp.sum(e2_ref[...] * oh_ref[...], axis=1, keepdims=True)
    hpre = jax.lax.dot_general(
        w1_ref[...].astype(jnp.bfloat16), esel.astype(jnp.bfloat16),
        (((1,), (0,)), ((), ())), preferred_element_type=jnp.float32)
    h_ref[...] = jnp.maximum(jnp.transpose(hpre, (1, 0)) + b1_ref[...], 0.0)


def _sc_tail(h, W2, b2):
    mesh = plsc.VectorSubcoreMesh(core_axis_name="c", subcore_axis_name="s")

    cp = pltpu.CompilerParams()
    if "needs_layout_passes" in pltpu.CompilerParams.__dataclass_fields__:
        cp = dataclasses.replace(cp, needs_layout_passes=False)

    @functools.partial(
        pl.kernel,
        out_type=jax.ShapeDtypeStruct((SC_ROWS,), jnp.float32),
        mesh=mesh,
        scratch_types=[pltpu.VMEM((HID,), jnp.float32)],
        compiler_params=cp,
    )
    def sc_kernel(h_hbm, w2_hbm, b2_hbm, out_hbm, h_v):
        pltpu.sync_copy(h_hbm.at[0], h_v)
        hs = [h_v[pl.ds(k * 16, 16)] for k in range(8)]

        lane16 = jax.lax.iota(jnp.int32, 16)

        def body(w_v, b_v, o_v):
            for g in range(SC_RB // 16):
                cur = b_v[pl.ds(g * 16, 16)]
                for l in range(16):
                    r = g * 16 + l
                    acc = w_v[r, pl.ds(0, 16)] * hs[0]
                    for k in range(1, 8):
                        acc += w_v[r, pl.ds(k * 16, 16)] * hs[k]
                    cur = jnp.where(lane16 == l, jnp.sum(acc) + cur, cur)
                o_v[pl.ds(g * 16, 16)] = cur

        pltpu.emit_pipeline(
            body,
            grid=(SC_STEPS,),
            in_specs=[
                pl.BlockSpec((SC_RB, HID), lambda i: (SPLIT // SC_RB + i, 0)),
                pl.BlockSpec((SC_RB,), lambda i: (SPLIT // SC_RB + i,)),
            ],
            out_specs=[pl.BlockSpec((SC_RB,), lambda i: (i,))],
            core_axis_name=("c", "s"),
            dimension_semantics=(pltpu.PARALLEL,),
        )(w2_hbm, b2_hbm, out_hbm)

    return sc_kernel(h, W2, b2)


def _pass_a_body(h_ref, w2_ref, b2_ref, out_ref, m_ref, s_ref,
                 mrun_ref, srun_ref):
    i = pl.program_id(0)

    @pl.when(i == 0)
    def _():
        mrun_ref[...] = jnp.full((1, HID), -1e30, jnp.float32)
        srun_ref[...] = jnp.zeros((1, HID), jnp.float32)

    w2b = w2_ref[...].astype(jnp.bfloat16)
    logits = jax.lax.dot_general(
        h_ref[...].astype(jnp.bfloat16), w2b, (((1,), (1,)), ((), ())),
        preferred_element_type=jnp.float32)
    row = logits + b2_ref[...][None, :]
    out_ref[...] = row
    mt = jnp.max(row)
    st = jnp.sum(jnp.exp(row - mt))
    mnew = jnp.maximum(mrun_ref[...], mt)
    srun_ref[...] = (srun_ref[...] * jnp.exp(mrun_ref[...] - mnew)
                     + st * jnp.exp(mt - mnew))
    mrun_ref[...] = mnew

    @pl.when(i == N_TC_TILES - 1)
    def _():
        m_ref[...] = mrun_ref[...]
        s_ref[...] = srun_ref[...]


def _pass_b_body(l_ref, lsc_ref, m_ref, s_ref, o_ref):
    ls = lsc_ref[...]
    ms = jnp.max(ls)
    ss = jnp.sum(jnp.exp(ls - ms))
    mtc = jnp.max(m_ref[...])
    stc = jnp.max(s_ref[...])
    big = jnp.maximum(mtc, ms)
    total = stc * jnp.exp(mtc - big) + ss * jnp.exp(ms - big)
    logz = big + jnp.log(total)
    o_ref[:, :SPLIT] = l_ref[:, :SPLIT] - logz
    o_ref[:, SPLIT:] = (ls - logz)[None, :]


def kernel(inputs, emb_table, W1, b1, W2, b2):
    oh = jax.nn.one_hot(inputs % 128, 128, dtype=jnp.bfloat16)
    oh_flat = jnp.repeat(oh, EMB_DIM, axis=0)
    b1r = b1.reshape(1, HID)

    h = pl.pallas_call(
        _prep_body,
        in_specs=[
            pl.BlockSpec(memory_space=pltpu.SMEM),
            pl.BlockSpec(memory_space=pltpu.MemorySpace.HBM),
            pl.BlockSpec(memory_space=pltpu.VMEM),
            pl.BlockSpec(memory_space=pltpu.VMEM),
            pl.BlockSpec(memory_space=pltpu.VMEM),
        ],
        out_specs=pl.BlockSpec(memory_space=pltpu.VMEM),
        out_shape=jax.ShapeDtypeStruct((1, HID), jnp.float32),
        scratch_shapes=[
            pltpu.VMEM((FLAT, 128), jnp.float32),
            pltpu.SemaphoreType.DMA,
        ],
    )(inputs, emb_table.T, oh_flat, W1, b1r)

    lsc = _sc_tail(h, W2, b2)

    logits, m, s = pl.pallas_call(
        _pass_a_body,
        grid=(N_TC_TILES,),
        in_specs=[
            pl.BlockSpec((1, HID), lambda i: (0, 0)),
            pl.BlockSpec((V_TILE, HID), lambda i: (i, 0)),
            pl.BlockSpec((V_TILE,), lambda i: (i,)),
        ],
        out_specs=[
            pl.BlockSpec((1, V_TILE), lambda i: (0, i)),
            pl.BlockSpec((1, HID), lambda i: (0, 0)),
            pl.BlockSpec((1, HID), lambda i: (0, 0)),
        ],
        out_shape=[
            jax.ShapeDtypeStruct((1, VOCAB_SIZE), jnp.float32),
            jax.ShapeDtypeStruct((1, HID), jnp.float32),
            jax.ShapeDtypeStruct((1, HID), jnp.float32),
        ],
        scratch_shapes=[
            pltpu.VMEM((1, HID), jnp.float32),
            pltpu.VMEM((1, HID), jnp.float32),
        ],
    )(h, W2, b2)

    out = pl.pallas_call(
        _pass_b_body,
        in_specs=[
            pl.BlockSpec(memory_space=pltpu.VMEM),
            pl.BlockSpec(memory_space=pltpu.VMEM),
            pl.BlockSpec(memory_space=pltpu.VMEM),
            pl.BlockSpec(memory_space=pltpu.VMEM),
        ],
        out_specs=pl.BlockSpec(memory_space=pltpu.VMEM),
        out_shape=jax.ShapeDtypeStruct((1, VOCAB_SIZE), jnp.float32),
    )(logits, lsc, m, s)

    return out


# speedup vs baseline: 1.0965x; 1.0965x over previous
import jax
import jax.numpy as jnp
from jax.experimental import pallas as pl
from jax.experimental.pallas import tpu as pltpu

VOCAB_SIZE = 1000000
EMB_DIM = 64
CTX = 200
HID = 128
FLAT = CTX * EMB_DIM

V_TILE = 20480
N_TILES = -(-VOCAB_SIZE // V_TILE)


def _pass_a_body(idx_ref, embt_ref, oh_ref, w1_ref, b1_ref, w2_ref, b2_ref,
                 out_ref, m_ref, s_ref, h_ref, e2_ref, mrun_ref, srun_ref, sem):
    i = pl.program_id(0)

    @pl.when(i == 0)
    def _():
        def issue(t, carry):
            r = idx_ref[t]
            start = pl.multiple_of((r // 128) * 128, 128)
            pltpu.make_async_copy(
                embt_ref.at[:, pl.ds(start, 128)],
                e2_ref.at[pl.ds(t * EMB_DIM, EMB_DIM), :],
                sem,
            ).start()
            return carry

        jax.lax.fori_loop(0, CTX, issue, 0)

        def drain(t, carry):
            pltpu.make_async_copy(
                embt_ref.at[:, pl.ds(0, 128)],
                e2_ref.at[pl.ds(t * EMB_DIM, EMB_DIM), :],
                sem,
            ).wait()
            return carry

        jax.lax.fori_loop(0, CTX, drain, 0)

        esel = jnp.sum(e2_ref[...] * oh_ref[...], axis=1, keepdims=True)
        hpre = jax.lax.dot_general(
            w1_ref[...].astype(jnp.bfloat16), esel.astype(jnp.bfloat16),
            (((1,), (0,)), ((), ())), preferred_element_type=jnp.float32)
        h = jnp.maximum(jnp.transpose(hpre, (1, 0)) + b1_ref[...], 0.0)
        h_ref[...] = h.astype(jnp.bfloat16)
        mrun_ref[...] = jnp.full((1, HID), -1e30, jnp.float32)
        srun_ref[...] = jnp.zeros((1, HID), jnp.float32)

    w2b = w2_ref[...].astype(jnp.bfloat16)
    logits = jax.lax.dot_general(
        h_ref[...], w2b, (((1,), (1,)), ((), ())),
        preferred_element_type=jnp.float32)
    row = logits + b2_ref[...][None, :]
    lane = jax.lax.broadcasted_iota(jnp.int32, (1, V_TILE), 1)
    row = jnp.where(lane < VOCAB_SIZE - i * V_TILE, row, -1e30)
    out_ref[...] = row
    mt = jnp.max(row)
    st = jnp.sum(jnp.exp(row - mt))
    mnew = jnp.maximum(mrun_ref[...], mt)
    srun_ref[...] = (srun_ref[...] * jnp.exp(mrun_ref[...] - mnew)
                     + st * jnp.exp(mt - mnew))
    mrun_ref[...] = mnew

    @pl.when(i == N_TILES - 1)
    def _():
        m_ref[...] = mrun_ref[...]
        s_ref[...] = srun_ref[...]


def _pass_b_body(l_ref, m_ref, s_ref, o_ref):
    logz = jnp.max(m_ref[...]) + jnp.log(jnp.max(s_ref[...]))
    o_ref[...] = l_ref[...] - logz


def kernel(inputs, emb_table, W1, b1, W2, b2):
    oh = jax.nn.one_hot(inputs % 128, 128, dtype=jnp.bfloat16)
    oh_flat = jnp.repeat(oh, EMB_DIM, axis=0)
    b1r = b1.reshape(1, HID)

    logits, m, s = pl.pallas_call(
        _pass_a_body,
        grid=(N_TILES,),
        in_specs=[
            pl.BlockSpec(memory_space=pltpu.SMEM),
            pl.BlockSpec(memory_space=pltpu.MemorySpace.HBM),
            pl.BlockSpec((FLAT, 128), lambda i: (0, 0)),
            pl.BlockSpec((HID, FLAT), lambda i: (0, 0)),
            pl.BlockSpec((1, HID), lambda i: (0, 0)),
            pl.BlockSpec((V_TILE, HID), lambda i: (i, 0)),
            pl.BlockSpec((V_TILE,), lambda i: (i,)),
        ],
        out_specs=[
            pl.BlockSpec((1, V_TILE), lambda i: (0, i)),
            pl.BlockSpec((1, HID), lambda i: (0, 0)),
            pl.BlockSpec((1, HID), lambda i: (0, 0)),
        ],
        out_shape=[
            jax.ShapeDtypeStruct((1, VOCAB_SIZE), jnp.float32),
            jax.ShapeDtypeStruct((1, HID), jnp.float32),
            jax.ShapeDtypeStruct((1, HID), jnp.float32),
        ],
        scratch_shapes=[
            pltpu.VMEM((1, HID), jnp.bfloat16),
            pltpu.VMEM((FLAT, 128), jnp.float32),
            pltpu.VMEM((1, HID), jnp.float32),
            pltpu.VMEM((1, HID), jnp.float32),
            pltpu.SemaphoreType.DMA,
        ],
    )(inputs, emb_table.T, oh_flat, W1, b1r, W2, b2)

    out = pl.pallas_call(
        _pass_b_body,
        in_specs=[
            pl.BlockSpec(memory_space=pltpu.VMEM),
            pl.BlockSpec(memory_space=pltpu.VMEM),
            pl.BlockSpec(memory_space=pltpu.VMEM),
        ],
        out_specs=pl.BlockSpec(memory_space=pltpu.VMEM),
        out_shape=jax.ShapeDtypeStruct((1, VOCAB_SIZE), jnp.float32),
    )(logits, m, s)

    return out
